# Initial kernel scaffold; baseline (speedup 1.0000x reference)
import jax
import jax.numpy as jnp
from jax.experimental import pallas as pl


def kernel(x, adj, W1, W2, W3, Wa1, Wa2, Wa3):
    raise NotImplementedError("write your pallas kernel here")


# speedup vs baseline: 1.2185x; 1.2185x over previous
import jax
import jax.numpy as jnp
from jax.experimental import pallas as pl


def _prep_kernel(x_ref, w1_ref, wa1_ref, wa2_ref, wa3_ref,
                 p_ref, mua_ref, logvara_ref):
    x = x_ref[...]
    p_ref[...] = jnp.dot(x, w1_ref[...], preferred_element_type=jnp.float32)
    ha1 = jnp.tanh(jax.lax.dot_general(
        x, wa1_ref[...], (((0,), (0,)), ((), ())),
        preferred_element_type=jnp.float32))
    mua_ref[...] = jnp.dot(ha1, wa2_ref[...],
                           preferred_element_type=jnp.float32)
    logvara_ref[...] = jnp.dot(ha1, wa3_ref[...],
                               preferred_element_type=jnp.float32)


def _adj_relu_kernel(adj_ref, p_ref, w23_ref, hw_ref):
    h = jnp.dot(adj_ref[...], p_ref[...], preferred_element_type=jnp.float32)
    h = jnp.maximum(h, 0.0)
    hw_ref[...] = jnp.dot(h, w23_ref[...], preferred_element_type=jnp.float32)


def _adj_plain_kernel(adj_ref, hw_ref, mu_ref, logvar_ref):
    ml = jnp.dot(adj_ref[...], hw_ref[...], preferred_element_type=jnp.float32)
    h2 = mu_ref.shape[1]
    mu_ref[...] = ml[:, :h2]
    logvar_ref[...] = ml[:, h2:]


def _dec_kernel(mu_blk_ref, mu_full_ref, mua_ref, adjrec_ref, xrec_ref):
    mu_i = mu_blk_ref[...]
    adjrec_ref[...] = jax.lax.dot_general(
        mu_i, mu_full_ref[...], (((1,), (1,)), ((), ())),
        preferred_element_type=jnp.float32)
    xrec_ref[...] = jax.lax.dot_general(
        mu_i, mua_ref[...], (((1,), (1,)), ((), ())),
        preferred_element_type=jnp.float32)


def kernel(x, adj, W1, W2, W3, Wa1, Wa2, Wa3):
    N, D = x.shape
    H1 = W1.shape[1]
    H2 = W2.shape[1]
    bm = 400 if N % 400 == 0 else 8
    grid = (N // bm,)

    P, mu_a, logvar_a = pl.pallas_call(
        _prep_kernel,
        out_shape=(
            jax.ShapeDtypeStruct((N, H1), jnp.float32),
            jax.ShapeDtypeStruct((D, H2), jnp.float32),
            jax.ShapeDtypeStruct((D, H2), jnp.float32),
        ),
    )(x, W1, Wa1, Wa2, Wa3)

    W23 = jnp.concatenate([W2, W3], axis=1)

    hw = pl.pallas_call(
        _adj_relu_kernel,
        grid=grid,
        in_specs=[
            pl.BlockSpec((bm, N), lambda i: (i, 0)),
            pl.BlockSpec((N, H1), lambda i: (0, 0)),
            pl.BlockSpec((H1, 2 * H2), lambda i: (0, 0)),
        ],
        out_specs=pl.BlockSpec((bm, 2 * H2), lambda i: (i, 0)),
        out_shape=jax.ShapeDtypeStruct((N, 2 * H2), jnp.float32),
    )(adj, P, W23)

    mu, logvar = pl.pallas_call(
        _adj_plain_kernel,
        grid=grid,
        in_specs=[
            pl.BlockSpec((bm, N), lambda i: (i, 0)),
            pl.BlockSpec((N, 2 * H2), lambda i: (0, 0)),
        ],
        out_specs=(pl.BlockSpec((bm, H2), lambda i: (i, 0)),
                   pl.BlockSpec((bm, H2), lambda i: (i, 0))),
        out_shape=(jax.ShapeDtypeStruct((N, H2), jnp.float32),
                   jax.ShapeDtypeStruct((N, H2), jnp.float32)),
    )(adj, hw)

    adj_rec, x_rec = pl.pallas_call(
        _dec_kernel,
        grid=grid,
        in_specs=[
            pl.BlockSpec((bm, H2), lambda i: (i, 0)),
            pl.BlockSpec((N, H2), lambda i: (0, 0)),
            pl.BlockSpec((D, H2), lambda i: (0, 0)),
        ],
        out_specs=(pl.BlockSpec((bm, N), lambda i: (i, 0)),
                   pl.BlockSpec((bm, D), lambda i: (i, 0))),
        out_shape=(jax.ShapeDtypeStruct((N, N), jnp.float32),
                   jax.ShapeDtypeStruct((N, D), jnp.float32)),
    )(mu, mu, mu_a)

    return (adj_rec, x_rec, mu, logvar, mu_a, logvar_a)


# speedup vs baseline: 1.2482x; 1.0243x over previous
import functools

import jax
import jax.numpy as jnp
from jax.experimental import pallas as pl
from jax.experimental.pallas import tpu as pltpu


def _mega_kernel(x_ref, adj_ref, w1_ref, w23_ref, wa1_ref, wa2_ref, wa3_ref,
                 adjrec_ref, xrec_ref, mu_ref, logvar_ref, mua_ref,
                 logvara_ref, p_sc, hw_sc, mu_sc, mua_sc, *, S, bm, H2):
    i = pl.program_id(0)

    @pl.when(i == 0)
    def _prep():
        xv = x_ref[...]
        p_sc[...] = jnp.dot(xv, w1_ref[...], preferred_element_type=jnp.float32)
        ha1 = jnp.tanh(jax.lax.dot_general(
            xv, wa1_ref[...], (((0,), (0,)), ((), ())),
            preferred_element_type=jnp.float32))
        mua = jnp.dot(ha1, wa2_ref[...], preferred_element_type=jnp.float32)
        mua_sc[...] = mua
        mua_ref[...] = mua
        logvara_ref[...] = jnp.dot(ha1, wa3_ref[...],
                                   preferred_element_type=jnp.float32)

    @pl.when(i < S)
    def _phase_a():
        h = jnp.maximum(
            jnp.dot(adj_ref[...], p_sc[...],
                    preferred_element_type=jnp.float32), 0.0)
        hw_sc[pl.ds(i * bm, bm), :] = jnp.dot(
            h, w23_ref[...], preferred_element_type=jnp.float32)

    @pl.when((i >= S) & (i < 2 * S))
    def _phase_b():
        ml = jnp.dot(adj_ref[...], hw_sc[...],
                     preferred_element_type=jnp.float32)
        mu_blk = ml[:, :H2]
        mu_ref[...] = mu_blk
        logvar_ref[...] = ml[:, H2:]
        mu_sc[pl.ds((i - S) * bm, bm), :] = mu_blk

    @pl.when(i >= 2 * S)
    def _phase_c():
        mu_i = mu_sc[pl.ds((i - 2 * S) * bm, bm), :]
        adjrec_ref[...] = jax.lax.dot_general(
            mu_i, mu_sc[...], (((1,), (1,)), ((), ())),
            preferred_element_type=jnp.float32)
        xrec_ref[...] = jax.lax.dot_general(
            mu_i, mua_sc[...], (((1,), (1,)), ((), ())),
            preferred_element_type=jnp.float32)


def kernel(x, adj, W1, W2, W3, Wa1, Wa2, Wa3):
    N, D = x.shape
    H1 = W1.shape[1]
    H2 = W2.shape[1]
    bm = 200 if N % 200 == 0 else 8
    S = N // bm

    W23 = jnp.concatenate([W2, W3], axis=1)

    def adj_idx(i):
        return (jnp.where(i < 2 * S, jax.lax.rem(i, S), S - 1), 0)

    def mu_idx(i):
        return (jnp.clip(i - S, 0, S - 1), 0)

    def dec_idx(i):
        return (jnp.clip(i - 2 * S, 0, S - 1), 0)

    zero2 = lambda i: (0, 0)

    adj_rec, x_rec, mu, logvar, mu_a, logvar_a = pl.pallas_call(
        functools.partial(_mega_kernel, S=S, bm=bm, H2=H2),
        grid=(3 * S,),
        in_specs=[
            pl.BlockSpec((N, D), zero2),
            pl.BlockSpec((bm, N), adj_idx),
            pl.BlockSpec((D, H1), zero2),
            pl.BlockSpec((H1, 2 * H2), zero2),
            pl.BlockSpec((N, H1), zero2),
            pl.BlockSpec((H1, H2), zero2),
            pl.BlockSpec((H1, H2), zero2),
        ],
        out_specs=(
            pl.BlockSpec((bm, N), dec_idx),
            pl.BlockSpec((bm, D), dec_idx),
            pl.BlockSpec((bm, H2), mu_idx),
            pl.BlockSpec((bm, H2), mu_idx),
            pl.BlockSpec((D, H2), zero2),
            pl.BlockSpec((D, H2), zero2),
        ),
        out_shape=(
            jax.ShapeDtypeStruct((N, N), jnp.float32),
            jax.ShapeDtypeStruct((N, D), jnp.float32),
            jax.ShapeDtypeStruct((N, H2), jnp.float32),
            jax.ShapeDtypeStruct((N, H2), jnp.float32),
            jax.ShapeDtypeStruct((D, H2), jnp.float32),
            jax.ShapeDtypeStruct((D, H2), jnp.float32),
        ),
        scratch_shapes=[
            pltpu.VMEM((N, H1), jnp.float32),
            pltpu.VMEM((N, 2 * H2), jnp.float32),
            pltpu.VMEM((N, H2), jnp.float32),
            pltpu.VMEM((D, H2), jnp.float32),
        ],
        compiler_params=pltpu.CompilerParams(
            dimension_semantics=("arbitrary",)),
    )(x, adj, W1, W23, Wa1, Wa2, Wa3)

    return (adj_rec, x_rec, mu, logvar, mu_a, logvar_a)


# speedup vs baseline: 1.2606x; 1.0099x over previous
import functools

import jax
import jax.numpy as jnp
from jax.experimental import pallas as pl
from jax.experimental.pallas import tpu as pltpu


def _enc_kernel(x_ref, adj_ref, w1_ref, w23_ref, wa1_ref, wa2_ref, wa3_ref,
                mu_ref, logvar_ref, mua_ref, logvara_ref,
                p_sc, hw_sc, *, S, bm, H2):
    i = pl.program_id(0)

    @pl.when(i == 0)
    def _prep():
        xv = x_ref[...]
        p_sc[...] = jnp.dot(xv, w1_ref[...], preferred_element_type=jnp.float32)
        ha1 = jnp.tanh(jax.lax.dot_general(
            xv, wa1_ref[...], (((0,), (0,)), ((), ())),
            preferred_element_type=jnp.float32))
        mua_ref[...] = jnp.dot(ha1, wa2_ref[...],
                               preferred_element_type=jnp.float32)
        logvara_ref[...] = jnp.dot(ha1, wa3_ref[...],
                                   preferred_element_type=jnp.float32)

    @pl.when(i < S)
    def _phase_a():
        h = jnp.maximum(
            jnp.dot(adj_ref[...], p_sc[...],
                    preferred_element_type=jnp.float32), 0.0)
        hw_sc[pl.ds(i * bm, bm), :] = jnp.dot(
            h, w23_ref[...], preferred_element_type=jnp.float32)

    @pl.when(i >= S)
    def _phase_b():
        ml = jnp.dot(adj_ref[...], hw_sc[...],
                     preferred_element_type=jnp.float32)
        mu_ref[...] = ml[:, :H2]
        logvar_ref[...] = ml[:, H2:]


def _dec_kernel(mu_blk_ref, mu_full_ref, mua_ref, adjrec_ref, xrec_ref):
    mu_i = mu_blk_ref[...]
    adjrec_ref[...] = jax.lax.dot_general(
        mu_i, mu_full_ref[...], (((1,), (1,)), ((), ())),
        preferred_element_type=jnp.float32)
    xrec_ref[...] = jax.lax.dot_general(
        mu_i, mua_ref[...], (((1,), (1,)), ((), ())),
        preferred_element_type=jnp.float32)


def kernel(x, adj, W1, W2, W3, Wa1, Wa2, Wa3):
    N, D = x.shape
    H1 = W1.shape[1]
    H2 = W2.shape[1]
    bm = 400 if N % 400 == 0 else 8
    S = N // bm

    W23 = jnp.concatenate([W2, W3], axis=1)

    zero2 = lambda i: (0, 0)

    mu, logvar, mu_a, logvar_a = pl.pallas_call(
        functools.partial(_enc_kernel, S=S, bm=bm, H2=H2),
        grid=(2 * S,),
        in_specs=[
            pl.BlockSpec((N, D), zero2),
            pl.BlockSpec((bm, N), lambda i: (jax.lax.rem(i, S), 0)),
            pl.BlockSpec((D, H1), zero2),
            pl.BlockSpec((H1, 2 * H2), zero2),
            pl.BlockSpec((N, H1), zero2),
            pl.BlockSpec((H1, H2), zero2),
            pl.BlockSpec((H1, H2), zero2),
        ],
        out_specs=(
            pl.BlockSpec((bm, H2), lambda i: (jnp.clip(i - S, 0, S - 1), 0)),
            pl.BlockSpec((bm, H2), lambda i: (jnp.clip(i - S, 0, S - 1), 0)),
            pl.BlockSpec((D, H2), zero2),
            pl.BlockSpec((D, H2), zero2),
        ),
        out_shape=(
            jax.ShapeDtypeStruct((N, H2), jnp.float32),
            jax.ShapeDtypeStruct((N, H2), jnp.float32),
            jax.ShapeDtypeStruct((D, H2), jnp.float32),
            jax.ShapeDtypeStruct((D, H2), jnp.float32),
        ),
        scratch_shapes=[
            pltpu.VMEM((N, H1), jnp.float32),
            pltpu.VMEM((N, 2 * H2), jnp.float32),
        ],
        compiler_params=pltpu.CompilerParams(
            dimension_semantics=("arbitrary",)),
    )(x, adj, W1, W23, Wa1, Wa2, Wa3)

    adj_rec, x_rec = pl.pallas_call(
        _dec_kernel,
        grid=(S,),
        in_specs=[
            pl.BlockSpec((bm, H2), lambda i: (i, 0)),
            pl.BlockSpec((N, H2), zero2),
            pl.BlockSpec((D, H2), zero2),
        ],
        out_specs=(
            pl.BlockSpec((bm, N), lambda i: (i, 0)),
            pl.BlockSpec((bm, D), lambda i: (i, 0)),
        ),
        out_shape=(
            jax.ShapeDtypeStruct((N, N), jnp.float32),
            jax.ShapeDtypeStruct((N, D), jnp.float32),
        ),
        compiler_params=pltpu.CompilerParams(
            dimension_semantics=("arbitrary",)),
    )(mu, mu, mu_a)

    return (adj_rec, x_rec, mu, logvar, mu_a, logvar_a)
